# Initial kernel scaffold; baseline (speedup 1.0000x reference)
import jax
import jax.numpy as jnp
from jax.experimental import pallas as pl


def kernel(raw_cls_bbox, raw_prob):
    raise NotImplementedError("write your pallas kernel here")


# speedup vs baseline: 123.6009x; 123.6009x over previous
import functools

import jax
import jax.numpy as jnp
from jax.experimental import pallas as pl
from jax.experimental.pallas import tpu as pltpu

_N_CLASS = 21
_R = 5000
_CM1 = _N_CLASS - 1
_NMS_THRESH = 0.3
_SCORE_THRESH = 0.05
_B = 512
_R_PAD = 5120


def _pair_iou(py1, px1, py2, px2, parea, cy1, cx1, cy2, cx2, carea):
    ih = jnp.maximum(
        jnp.minimum(py2[:, None], cy2[None, :]) - jnp.maximum(py1[:, None], cy1[None, :]),
        0.0,
    )
    iw = jnp.maximum(
        jnp.minimum(px2[:, None], cx2[None, :]) - jnp.maximum(px1[:, None], cx1[None, :]),
        0.0,
    )
    inter = ih * iw
    return inter / (parea[:, None] + carea[None, :] - inter + 1e-9)


def _nms_body(nb_ref, data_ref, keep_ref):
    c = pl.program_id(0)
    nb = nb_ref[c]

    keep_ref[0, 0, :] = jnp.zeros((_R_PAD,), jnp.float32)

    def block_body(i, carry):
        base = i * _B
        cy1 = data_ref[0, 0, pl.ds(base, _B)]
        cx1 = data_ref[0, 1, pl.ds(base, _B)]
        cy2 = data_ref[0, 2, pl.ds(base, _B)]
        cx2 = data_ref[0, 3, pl.ds(base, _B)]
        cval = data_ref[0, 4, pl.ds(base, _B)]
        carea = jnp.maximum(cy2 - cy1, 0.0) * jnp.maximum(cx2 - cx1, 0.0)

        def prev_body(j, ext):
            pb = j * _B
            py1 = data_ref[0, 0, pl.ds(pb, _B)]
            px1 = data_ref[0, 1, pl.ds(pb, _B)]
            py2 = data_ref[0, 2, pl.ds(pb, _B)]
            px2 = data_ref[0, 3, pl.ds(pb, _B)]
            pkeep = keep_ref[0, 0, pl.ds(pb, _B)]
            parea = jnp.maximum(py2 - py1, 0.0) * jnp.maximum(px2 - px1, 0.0)
            iou = _pair_iou(py1, px1, py2, px2, parea, cy1, cx1, cy2, cx2, carea)
            sup = jnp.where(iou > _NMS_THRESH, pkeep[:, None], 0.0)
            return jnp.maximum(ext, jnp.max(sup, axis=0))

        ext_sup = jax.lax.fori_loop(0, i, prev_body, jnp.zeros((_B,), jnp.float32))

        iou_bb = _pair_iou(cy1, cx1, cy2, cx2, carea, cy1, cx1, cy2, cx2, carea)
        rows = jax.lax.broadcasted_iota(jnp.int32, (_B, _B), 0)
        cols = jax.lax.broadcasted_iota(jnp.int32, (_B, _B), 1)
        msup = jnp.where((iou_bb > _NMS_THRESH) & (rows < cols), 1.0, 0.0)

        supp0 = jnp.maximum(1.0 - cval, ext_sup)
        keep0 = jnp.zeros((_B,), jnp.float32)
        und0 = cval * (1.0 - supp0)

        def peel_cond(state):
            _, _, und = state
            return jnp.max(und) > 0.0

        def peel_body(state):
            keep, supp, und = state
            pot = jnp.max(msup * (1.0 - supp)[:, None], axis=0)
            new_keep = und * (1.0 - pot)
            keep = jnp.maximum(keep, new_keep)
            new_supp = und * jnp.max(msup * keep[:, None], axis=0)
            supp = jnp.maximum(supp, new_supp)
            und = und * (1.0 - keep) * (1.0 - supp)
            return keep, supp, und

        keep, _, _ = jax.lax.while_loop(peel_cond, peel_body, (keep0, supp0, und0))
        keep_ref[0, 0, pl.ds(base, _B)] = keep
        return carry

    jax.lax.fori_loop(0, nb, block_body, 0, unroll=False)


@functools.partial(jax.jit)
def kernel(raw_cls_bbox, raw_prob):
    boxes_cls = jnp.transpose(
        raw_cls_bbox.reshape(_R, _N_CLASS, 4)[:, 1:, :], (1, 0, 2)
    )
    probs_cls = jnp.transpose(raw_prob[:, 1:], (1, 0))

    valid = probs_cls > _SCORE_THRESH
    order = jnp.argsort(-jnp.where(valid, probs_cls, -jnp.inf), axis=1)
    sb = jnp.take_along_axis(boxes_cls, order[:, :, None], axis=1)
    sv = jnp.take_along_axis(valid, order, axis=1)

    nvalid = jnp.sum(valid.astype(jnp.int32), axis=1)
    nblocks = (nvalid + (_B - 1)) // _B

    comp = jnp.transpose(sb, (0, 2, 1))
    data = jnp.concatenate(
        [comp, sv.astype(jnp.float32)[:, None, :], jnp.zeros((_CM1, 3, _R), jnp.float32)],
        axis=1,
    )
    data = jnp.pad(data, ((0, 0), (0, 0), (0, _R_PAD - _R)))

    grid_spec = pltpu.PrefetchScalarGridSpec(
        num_scalar_prefetch=1,
        grid=(_CM1,),
        in_specs=[pl.BlockSpec((1, 8, _R_PAD), lambda c, nb: (c, 0, 0))],
        out_specs=pl.BlockSpec((1, 1, _R_PAD), lambda c, nb: (c, 0, 0)),
    )
    keep_sorted = pl.pallas_call(
        _nms_body,
        grid_spec=grid_spec,
        out_shape=jax.ShapeDtypeStruct((_CM1, 1, _R_PAD), jnp.float32),
    )(nblocks, data)

    keep_sorted = keep_sorted[:, 0, :_R]
    m = jax.vmap(lambda o, k: jnp.zeros((_R,), jnp.float32).at[o].set(k))(
        order, keep_sorted
    )

    out = jnp.concatenate(
        [boxes_cls * m[:, :, None], (probs_cls * m)[:, :, None]], axis=-1
    )
    return out


# speedup vs baseline: 197.1332x; 1.5949x over previous
import functools

import jax
import jax.numpy as jnp
from jax.experimental import pallas as pl
from jax.experimental.pallas import tpu as pltpu

_N_CLASS = 21
_R = 5000
_CM1 = _N_CLASS - 1
_NMS_THRESH = 0.3
_SCORE_THRESH = 0.05
_B = 512
_R_PAD = 5120


def _pair_iou(py1, px1, py2, px2, parea, cy1, cx1, cy2, cx2, carea):
    ih = jnp.maximum(
        jnp.minimum(py2[:, None], cy2[None, :]) - jnp.maximum(py1[:, None], cy1[None, :]),
        0.0,
    )
    iw = jnp.maximum(
        jnp.minimum(px2[:, None], cx2[None, :]) - jnp.maximum(px1[:, None], cx1[None, :]),
        0.0,
    )
    inter = ih * iw
    return inter / (parea[:, None] + carea[None, :] - inter + 1e-9)


def _nms_body(nb_ref, data_ref, keep_ref):
    c = pl.program_id(0)
    nb = nb_ref[c]

    keep_ref[0, 0, :] = jnp.zeros((_R_PAD,), jnp.float32)

    def block_body(i, carry):
        base = i * _B
        cy1 = data_ref[0, 0, pl.ds(base, _B)]
        cx1 = data_ref[0, 1, pl.ds(base, _B)]
        cy2 = data_ref[0, 2, pl.ds(base, _B)]
        cx2 = data_ref[0, 3, pl.ds(base, _B)]
        cval = data_ref[0, 4, pl.ds(base, _B)]
        carea = jnp.maximum(cy2 - cy1, 0.0) * jnp.maximum(cx2 - cx1, 0.0)

        def prev_body(j, ext):
            pb = j * _B
            py1 = data_ref[0, 0, pl.ds(pb, _B)]
            px1 = data_ref[0, 1, pl.ds(pb, _B)]
            py2 = data_ref[0, 2, pl.ds(pb, _B)]
            px2 = data_ref[0, 3, pl.ds(pb, _B)]
            pkeep = keep_ref[0, 0, pl.ds(pb, _B)]
            parea = jnp.maximum(py2 - py1, 0.0) * jnp.maximum(px2 - px1, 0.0)
            iou = _pair_iou(py1, px1, py2, px2, parea, cy1, cx1, cy2, cx2, carea)
            sup = jnp.where(iou > _NMS_THRESH, pkeep[:, None], 0.0)
            return jnp.maximum(ext, jnp.max(sup, axis=0))

        ext_sup = jax.lax.fori_loop(0, i, prev_body, jnp.zeros((_B,), jnp.float32))

        iou_bb = _pair_iou(cy1, cx1, cy2, cx2, carea, cy1, cx1, cy2, cx2, carea)
        rows = jax.lax.broadcasted_iota(jnp.int32, (_B, _B), 0)
        cols = jax.lax.broadcasted_iota(jnp.int32, (_B, _B), 1)
        msup = jnp.where((iou_bb > _NMS_THRESH) & (rows < cols), 1.0, 0.0)

        supp0 = jnp.maximum(1.0 - cval, ext_sup)
        keep0 = jnp.zeros((_B,), jnp.float32)
        und0 = cval * (1.0 - supp0)

        def peel_once(state):
            keep, supp, und = state
            pot = jnp.max(msup * (1.0 - supp)[:, None], axis=0)
            new_keep = und * (1.0 - pot)
            keep = jnp.maximum(keep, new_keep)
            new_supp = und * jnp.max(msup * keep[:, None], axis=0)
            supp = jnp.maximum(supp, new_supp)
            und = und * (1.0 - keep) * (1.0 - supp)
            return keep, supp, und

        def peel_cond(state):
            return jnp.max(state[2]) > 0.0

        def peel4(state):
            for _ in range(4):
                state = peel_once(state)
            return state

        keep, _, _ = jax.lax.while_loop(
            peel_cond, peel4, peel4((keep0, supp0, und0))
        )
        keep_ref[0, 0, pl.ds(base, _B)] = keep
        return carry

    jax.lax.fori_loop(0, nb, block_body, 0, unroll=False)


@functools.partial(jax.jit)
def kernel(raw_cls_bbox, raw_prob):
    boxes_cls = jnp.transpose(
        raw_cls_bbox.reshape(_R, _N_CLASS, 4)[:, 1:, :], (1, 0, 2)
    )
    probs_cls = jnp.transpose(raw_prob[:, 1:], (1, 0))

    valid = probs_cls > _SCORE_THRESH
    order = jnp.argsort(-jnp.where(valid, probs_cls, -jnp.inf), axis=1)
    sb = jnp.take_along_axis(boxes_cls, order[:, :, None], axis=1)
    sv = jnp.take_along_axis(valid, order, axis=1)

    nvalid = jnp.sum(valid.astype(jnp.int32), axis=1)
    nblocks = (nvalid + (_B - 1)) // _B

    comp = jnp.transpose(sb, (0, 2, 1))
    data = jnp.concatenate(
        [comp, sv.astype(jnp.float32)[:, None, :], jnp.zeros((_CM1, 3, _R), jnp.float32)],
        axis=1,
    )
    data = jnp.pad(data, ((0, 0), (0, 0), (0, _R_PAD - _R)))

    grid_spec = pltpu.PrefetchScalarGridSpec(
        num_scalar_prefetch=1,
        grid=(_CM1,),
        in_specs=[pl.BlockSpec((1, 8, _R_PAD), lambda c, nb: (c, 0, 0))],
        out_specs=pl.BlockSpec((1, 1, _R_PAD), lambda c, nb: (c, 0, 0)),
    )
    keep_sorted = pl.pallas_call(
        _nms_body,
        grid_spec=grid_spec,
        out_shape=jax.ShapeDtypeStruct((_CM1, 1, _R_PAD), jnp.float32),
    )(nblocks, data)

    keep_sorted = keep_sorted[:, 0, :_R]
    inv = jnp.argsort(order, axis=1)
    m = jnp.take_along_axis(keep_sorted, inv, axis=1)

    out = jnp.concatenate(
        [boxes_cls * m[:, :, None], (probs_cls * m)[:, :, None]], axis=-1
    )
    return out
